# Initial kernel scaffold; baseline (speedup 1.0000x reference)
import jax
import jax.numpy as jnp
from jax.experimental import pallas as pl


def kernel(support_features, support_mask, global_prototypes, canonical_queries, appearance_queries, gat_W, gat_a, se_fc1, se_fc2):
    raise NotImplementedError("write your pallas kernel here")


# speedup vs baseline: 4.6397x; 4.6397x over previous
import functools

import jax
import jax.numpy as jnp
from jax.experimental import pallas as pl
from jax.experimental.pallas import tpu as pltpu

K = 8
MPK = 4
C = 256
H = 22
W = 22
KN = 10
TOPN = 32
ALPHA = 0.2
N = H * W
NP = 512
NEG = -1e30


def _rownorm(x):
    n = jnp.sqrt(jnp.sum(x * x, axis=-1, keepdims=True))
    return x / jnp.maximum(n, 1e-12)


def _kth_threshold(scores, k):
    def body(_, w):
        cur = jnp.max(w, axis=1, keepdims=True)
        return jnp.where(w == cur, NEG, w)
    work = jax.lax.fori_loop(0, k - 1, body, scores)
    return jnp.max(work, axis=1, keepdims=True)


def _masked_softmax(logits):
    m = jnp.max(logits, axis=1, keepdims=True)
    p = jnp.exp(logits - m)
    return p / jnp.sum(p, axis=1, keepdims=True)


def _gap_body(nodes_ref, cq_ref, aq_ref, gw_ref, amat_ref, fc1_ref, fc2_ref,
              canon_ref, app_ref):
    nodes = nodes_ref[...]
    col_valid = jax.lax.broadcasted_iota(jnp.int32, (1, NP), 1) < N
    row_valid = jax.lax.broadcasted_iota(jnp.int32, (NP, 1), 0) < N

    nn = _rownorm(nodes)
    sim = jax.lax.dot_general(nn, nn, (((1,), (1,)), ((), ())),
                              preferred_element_type=jnp.float32)
    sim_m = jnp.where(col_valid & row_valid, sim, NEG)
    thr = _kth_threshold(sim_m, KN)
    adj_bin = (sim_m >= thr) & row_valid & col_valid
    bin_sym = (adj_bin | adj_bin.T).astype(jnp.float32)
    adj = bin_sym * sim

    Wh = jax.lax.dot_general(nodes, gw_ref[...], (((1,), (1,)), ((), ())),
                             preferred_element_type=jnp.float32)
    fs = jnp.dot(Wh, amat_ref[...], preferred_element_type=jnp.float32)
    f1 = fs[:, 0:1]
    f2 = fs[:, 1:2]
    e_pre = f1 + f2.T
    e = jnp.where(e_pre >= 0, e_pre, ALPHA * e_pre)
    att_pre = jnp.where(adj > 0, e, jnp.float32(-9e15)) * adj
    att_pre = jnp.where(col_valid, att_pre, NEG)
    att = _masked_softmax(att_pre)
    h_prime = jnp.dot(att, Wh, preferred_element_type=jnp.float32)

    z = (nodes + h_prime) * row_valid.astype(jnp.float32)
    zmean = jnp.sum(z, axis=0, keepdims=True) / jnp.float32(N)
    t1 = jnp.dot(zmean, fc1_ref[...].T, preferred_element_type=jnp.float32)
    t1 = jnp.maximum(t1, 0.0)
    t2 = jnp.dot(t1, fc2_ref[...].T, preferred_element_type=jnp.float32)
    y = jax.nn.sigmoid(t2)
    refined = z * y

    cqn = _rownorm(cq_ref[...])
    rn = _rownorm(refined)
    aff = jax.lax.dot_general(cqn, rn, (((1,), (1,)), ((), ())),
                              preferred_element_type=jnp.float32)
    aff_m = jnp.where(col_valid, aff, NEG)
    canon_ref[...] = jnp.dot(_masked_softmax(aff_m), refined,
                             preferred_element_type=jnp.float32)

    thr32 = _kth_threshold(aff_m, TOPN)
    sel = aff_m >= thr32
    aqn = _rownorm(aq_ref[...])
    la = jax.lax.dot_general(aqn, rn, (((1,), (1,)), ((), ())),
                             preferred_element_type=jnp.float32)
    sel_rep = jnp.broadcast_to(sel[:, None, :], (K, MPK, NP)).reshape(K * MPK, NP)
    la_rep = jnp.broadcast_to(la[None, :, :], (K, MPK, NP)).reshape(K * MPK, NP)
    logits = jnp.where(sel_rep, la_rep, NEG)
    app_ref[...] = jnp.dot(_masked_softmax(logits), refined,
                           preferred_element_type=jnp.float32)


@functools.partial(jax.jit, static_argnames=("interpret",))
def _run(nodes_pad, cq, aq, gat_W, a_mat, se_fc1, se_fc2, interpret=False):
    canon, app = pl.pallas_call(
        _gap_body,
        out_shape=[
            jax.ShapeDtypeStruct((K, C), jnp.float32),
            jax.ShapeDtypeStruct((K * MPK, C), jnp.float32),
        ],
        interpret=interpret,
    )(nodes_pad, cq, aq, gat_W, a_mat, se_fc1, se_fc2)
    return canon, app


def kernel(support_features, support_mask, global_prototypes, canonical_queries,
           appearance_queries, gat_W, gat_a, se_fc1, se_fc2, *, interpret=False):
    nodes = support_features[0].reshape(C, N).T
    nodes_pad = jnp.pad(nodes, ((0, NP - N), (0, 0)))
    a_mat = jnp.stack([gat_a[0, :C], gat_a[0, C:]], axis=1)
    canon, app = _run(nodes_pad, canonical_queries, appearance_queries,
                      gat_W, a_mat, se_fc1, se_fc2, interpret=interpret)
    return (canon[None], app[None], jnp.float32(0.0))


# speedup vs baseline: 5.1408x; 1.1080x over previous
import functools

import jax
import jax.numpy as jnp
from jax.experimental import pallas as pl
from jax.experimental.pallas import tpu as pltpu

K = 8
MPK = 4
C = 256
H = 22
W = 22
KN = 10
TOPN = 32
ALPHA = 0.2
N = H * W
NP = 512
NEG = -1e30


def _rownorm(x):
    n = jnp.sqrt(jnp.sum(x * x, axis=-1, keepdims=True))
    return x / jnp.maximum(n, 1e-12)


def _kth_threshold(scores, k):
    m = jnp.max(scores, axis=1, keepdims=True)
    for _ in range(k - 1):
        m = jnp.max(jnp.where(scores < m, scores, NEG), axis=1, keepdims=True)
    return m


def _masked_softmax(logits):
    m = jnp.max(logits, axis=1, keepdims=True)
    p = jnp.exp(logits - m)
    return p / jnp.sum(p, axis=1, keepdims=True)


def _gap_body(nodes_ref, cq_ref, aq_ref, gw_ref, amat_ref, fc1_ref, fc2_ref,
              canon_ref, app_ref):
    nodes = nodes_ref[...]
    col_valid = jax.lax.broadcasted_iota(jnp.int32, (1, NP), 1) < N
    row_valid = jax.lax.broadcasted_iota(jnp.int32, (NP, 1), 0) < N

    nn = _rownorm(nodes)
    sim = jax.lax.dot_general(nn, nn, (((1,), (1,)), ((), ())),
                              preferred_element_type=jnp.float32)
    sim_m = jnp.where(col_valid & row_valid, sim, NEG)
    thr = _kth_threshold(sim_m, KN)
    adj_bin = (sim_m >= thr) & row_valid & col_valid
    bin_sym = (adj_bin | adj_bin.T).astype(jnp.float32)
    adj = bin_sym * sim

    Wh = jax.lax.dot_general(nodes, gw_ref[...], (((1,), (1,)), ((), ())),
                             preferred_element_type=jnp.float32)
    fs = jnp.dot(Wh, amat_ref[...], preferred_element_type=jnp.float32)
    f1 = fs[:, 0:1]
    f2 = fs[:, 1:2]
    e_pre = f1 + f2.T
    e = jnp.where(e_pre >= 0, e_pre, ALPHA * e_pre)
    att_pre = jnp.where(adj > 0, e, jnp.float32(-9e15)) * adj
    att_pre = jnp.where(col_valid, att_pre, NEG)
    att = _masked_softmax(att_pre)
    h_prime = jnp.dot(att, Wh, preferred_element_type=jnp.float32)

    z = (nodes + h_prime) * row_valid.astype(jnp.float32)
    zmean = jnp.sum(z, axis=0, keepdims=True) / jnp.float32(N)
    t1 = jnp.dot(zmean, fc1_ref[...].T, preferred_element_type=jnp.float32)
    t1 = jnp.maximum(t1, 0.0)
    t2 = jnp.dot(t1, fc2_ref[...].T, preferred_element_type=jnp.float32)
    y = jax.nn.sigmoid(t2)
    refined = z * y

    cqn = _rownorm(cq_ref[...])
    rn = _rownorm(refined)
    aff = jax.lax.dot_general(cqn, rn, (((1,), (1,)), ((), ())),
                              preferred_element_type=jnp.float32)
    aff_m = jnp.where(col_valid, aff, NEG)
    canon_ref[...] = jnp.dot(_masked_softmax(aff_m), refined,
                             preferred_element_type=jnp.float32)

    thr32 = _kth_threshold(aff_m, TOPN)
    sel = aff_m >= thr32
    aqn = _rownorm(aq_ref[...])
    la = jax.lax.dot_general(aqn, rn, (((1,), (1,)), ((), ())),
                             preferred_element_type=jnp.float32)
    sel_rep = jnp.broadcast_to(sel[:, None, :], (K, MPK, NP)).reshape(K * MPK, NP)
    la_rep = jnp.broadcast_to(la[None, :, :], (K, MPK, NP)).reshape(K * MPK, NP)
    logits = jnp.where(sel_rep, la_rep, NEG)
    app_ref[...] = jnp.dot(_masked_softmax(logits), refined,
                           preferred_element_type=jnp.float32)


@functools.partial(jax.jit, static_argnames=("interpret",))
def _run(nodes_pad, cq, aq, gat_W, a_mat, se_fc1, se_fc2, interpret=False):
    canon, app = pl.pallas_call(
        _gap_body,
        out_shape=[
            jax.ShapeDtypeStruct((K, C), jnp.float32),
            jax.ShapeDtypeStruct((K * MPK, C), jnp.float32),
        ],
        interpret=interpret,
    )(nodes_pad, cq, aq, gat_W, a_mat, se_fc1, se_fc2)
    return canon, app


def kernel(support_features, support_mask, global_prototypes, canonical_queries,
           appearance_queries, gat_W, gat_a, se_fc1, se_fc2, *, interpret=False):
    nodes = support_features[0].reshape(C, N).T
    nodes_pad = jnp.pad(nodes, ((0, NP - N), (0, 0)))
    a_mat = jnp.stack([gat_a[0, :C], gat_a[0, C:]], axis=1)
    canon, app = _run(nodes_pad, canonical_queries, appearance_queries,
                      gat_W, a_mat, se_fc1, se_fc2, interpret=interpret)
    return (canon[None], app[None], jnp.float32(0.0))


# speedup vs baseline: 5.3424x; 1.0392x over previous
import functools

import jax
import jax.numpy as jnp
from jax.experimental import pallas as pl
from jax.experimental.pallas import tpu as pltpu

K = 8
MPK = 4
C = 256
H = 22
W = 22
KN = 10
TOPN = 32
ALPHA = 0.2
N = H * W
NP = 512
NEG = -1e30


def _rownorm(x):
    n = jnp.sqrt(jnp.sum(x * x, axis=-1, keepdims=True))
    return x / jnp.maximum(n, 1e-12)


def _kth_threshold(scores, k):
    m = jnp.max(scores, axis=1, keepdims=True)
    for _ in range(k - 1):
        m = jnp.max(jnp.where(scores < m, scores, NEG), axis=1, keepdims=True)
    return m


def _masked_softmax(logits):
    m = jnp.max(logits, axis=1, keepdims=True)
    p = jnp.exp(logits - m)
    return p / jnp.sum(p, axis=1, keepdims=True)


def _gap_body(nodes_ref, cq_ref, aq_ref, gw_ref, amat_ref, fc1_ref, fc2_ref,
              canon_ref, app_ref):
    nodes = nodes_ref[...]
    col_iota = jax.lax.broadcasted_iota(jnp.int32, (1, NP), 1)
    colbias = jnp.where(col_iota < N, 0.0, NEG)
    row_valid = jax.lax.broadcasted_iota(jnp.int32, (NP, 1), 0) < N

    nn = _rownorm(nodes)
    sim = jax.lax.dot_general(nn, nn, (((1,), (1,)), ((), ())),
                              preferred_element_type=jnp.float32)
    sim_m = sim + colbias
    thr = _kth_threshold(sim_m, KN)
    thr_row = thr.reshape(1, NP)
    bin_sym = ((sim_m >= thr) & row_valid) | \
              ((sim_m >= thr_row) & (col_iota < N))

    Wh = jax.lax.dot_general(nodes, gw_ref[...], (((1,), (1,)), ((), ())),
                             preferred_element_type=jnp.float32)
    fs = jnp.dot(Wh, amat_ref[...], preferred_element_type=jnp.float32)
    f1 = fs[:, 0:1]
    f2 = fs[:, 1:2]
    e_pre = f1 + f2.T
    e = jnp.where(e_pre >= 0, e_pre, ALPHA * e_pre)
    att_pre = jnp.where(bin_sym,
                        jnp.where(sim > 0, e, jnp.float32(-9e15)) * sim,
                        0.0) + colbias
    m = jnp.max(att_pre, axis=1, keepdims=True)
    p = jnp.exp(att_pre - m)
    denom = jnp.sum(p, axis=1, keepdims=True)
    h_prime = jnp.dot(p, Wh, preferred_element_type=jnp.float32) / denom

    z = (nodes + h_prime) * row_valid.astype(jnp.float32)
    zmean = jnp.sum(z, axis=0, keepdims=True) / jnp.float32(N)
    t1 = jnp.dot(zmean, fc1_ref[...].T, preferred_element_type=jnp.float32)
    t1 = jnp.maximum(t1, 0.0)
    t2 = jnp.dot(t1, fc2_ref[...].T, preferred_element_type=jnp.float32)
    y = jax.nn.sigmoid(t2)
    refined = z * y

    cqn = _rownorm(cq_ref[...])
    rn = _rownorm(refined)
    aff = jax.lax.dot_general(cqn, rn, (((1,), (1,)), ((), ())),
                              preferred_element_type=jnp.float32)
    aff_m = aff + colbias
    canon_ref[...] = jnp.dot(_masked_softmax(aff_m), refined,
                             preferred_element_type=jnp.float32)

    thr32 = _kth_threshold(aff_m, TOPN)
    sel = aff_m >= thr32
    aqn = _rownorm(aq_ref[...])
    la = jax.lax.dot_general(aqn, rn, (((1,), (1,)), ((), ())),
                             preferred_element_type=jnp.float32)
    sel_rep = jnp.broadcast_to(sel[:, None, :], (K, MPK, NP)).reshape(K * MPK, NP)
    la_rep = jnp.broadcast_to(la[None, :, :], (K, MPK, NP)).reshape(K * MPK, NP)
    logits = jnp.where(sel_rep, la_rep, NEG)
    app_ref[...] = jnp.dot(_masked_softmax(logits), refined,
                           preferred_element_type=jnp.float32)


@functools.partial(jax.jit, static_argnames=("interpret",))
def _run(nodes_pad, cq, aq, gat_W, a_mat, se_fc1, se_fc2, interpret=False):
    canon, app = pl.pallas_call(
        _gap_body,
        out_shape=[
            jax.ShapeDtypeStruct((K, C), jnp.float32),
            jax.ShapeDtypeStruct((K * MPK, C), jnp.float32),
        ],
        interpret=interpret,
    )(nodes_pad, cq, aq, gat_W, a_mat, se_fc1, se_fc2)
    return canon, app


def kernel(support_features, support_mask, global_prototypes, canonical_queries,
           appearance_queries, gat_W, gat_a, se_fc1, se_fc2, *, interpret=False):
    nodes = support_features[0].reshape(C, N).T
    nodes_pad = jnp.pad(nodes, ((0, NP - N), (0, 0)))
    a_mat = jnp.stack([gat_a[0, :C], gat_a[0, C:]], axis=1)
    canon, app = _run(nodes_pad, canonical_queries, appearance_queries,
                      gat_W, a_mat, se_fc1, se_fc2, interpret=interpret)
    return (canon[None], app[None], jnp.float32(0.0))
